# baseline (speedup 1.0000x reference)
import functools
import math

import jax
import jax.numpy as jnp
from jax import lax
from jax.experimental import pallas as pl
from jax.experimental.pallas import tpu as pltpu
from jax.experimental.pallas import tpu_sc as plsc

D_MODEL = 64
SCALE = math.sqrt(D_MODEL)
NUM_WORKERS = 32
SEQ = 200
BATCH = 4096
BLK = 128
LANES = 16


@functools.partial(
    pl.kernel,
    mesh=plsc.VectorSubcoreMesh(core_axis_name="c", subcore_axis_name="s"),
    out_type=jax.ShapeDtypeStruct((SEQ, 8, NUM_WORKERS, 8, 128), jnp.float32),
    scratch_types=[
        pltpu.VMEM((SEQ, BLK), jnp.int32),
        pltpu.VMEM((BLK, D_MODEL), jnp.float32),
        pltpu.VMEM((BLK, D_MODEL), jnp.float32),
        pltpu.VMEM((8, 8, 128), jnp.float32),
        pltpu.SemaphoreType.DMA,
        pltpu.SemaphoreType.DMA,
    ],
    compiler_params=pltpu.CompilerParams(
        use_tc_tiling_on_sc=False, needs_layout_passes=False
    ),
)
def _emb_lookup(xt_hbm, lut_hbm, out_hbm, idx_v, rows0, rows1, tbuf, g0, g1):
    w = lax.axis_index("s") * 2 + lax.axis_index("c")

    pltpu.sync_copy(xt_hbm.at[:, pl.ds(w * BLK, BLK)], idx_v)

    def start_gather(s, buf, gsem):
        pltpu.make_async_copy(lut_hbm.at[idx_v.at[s]], buf, gsem).start()

    def wait_gather(buf, gsem):
        pltpu.make_async_copy(lut_hbm.at[idx_v.at[0]], buf, gsem).wait()

    iota = lax.iota(jnp.int32, LANES)
    row_sel = [iota + lg * LANES for lg in range(8)]

    def transpose_scale(rows):
        def col_body(d, _):
            dt = d // 8
            ds = d % 8
            col = jnp.full((LANES,), d, jnp.int32)
            for lg in range(8):
                v = plsc.load_gather(rows, [row_sel[lg], col])
                tbuf[dt, ds, pl.ds(lg * LANES, LANES)] = v * SCALE
            return 0

        lax.fori_loop(0, D_MODEL, col_body, 0, unroll=2)

    def write_out(s):
        pltpu.sync_copy(tbuf, out_hbm.at[s, :, w])

    start_gather(0, rows0, g0)
    start_gather(1, rows1, g1)

    def pair_body(k, _):
        s0 = 2 * k
        wait_gather(rows0, g0)
        transpose_scale(rows0)
        start_gather(s0 + 2, rows0, g0)
        write_out(s0)

        wait_gather(rows1, g1)
        transpose_scale(rows1)
        start_gather(s0 + 3, rows1, g1)
        write_out(s0 + 1)
        return 0

    lax.fori_loop(0, SEQ // 2 - 1, pair_body, 0)

    wait_gather(rows0, g0)
    transpose_scale(rows0)
    write_out(SEQ - 2)
    wait_gather(rows1, g1)
    transpose_scale(rows1)
    write_out(SEQ - 1)


def kernel(x, lut):
    xt = jnp.transpose(x).astype(jnp.int32)
    out_phys = _emb_lookup(xt, lut)
    return jnp.transpose(out_phys, (2, 4, 0, 1, 3)).reshape(BATCH, SEQ, D_MODEL)


# speedup vs baseline: 1.5726x; 1.5726x over previous
import functools
import math

import jax
import jax.numpy as jnp
from jax import lax
from jax.experimental import pallas as pl
from jax.experimental.pallas import tpu as pltpu
from jax.experimental.pallas import tpu_sc as plsc

D_MODEL = 64
SCALE = math.sqrt(D_MODEL)
NUM_WORKERS = 32
SEQ = 200
BATCH = 4096
BLK = 128
LANES = 16


@functools.partial(
    pl.kernel,
    mesh=plsc.VectorSubcoreMesh(core_axis_name="c", subcore_axis_name="s"),
    out_type=jax.ShapeDtypeStruct((SEQ, 8, NUM_WORKERS, 8, 128), jnp.float32),
    scratch_types=[
        pltpu.VMEM((SEQ, BLK), jnp.int32),
        pltpu.VMEM((BLK, D_MODEL), jnp.float32),
        pltpu.VMEM((BLK, D_MODEL), jnp.float32),
        pltpu.VMEM((8, 8, 128), jnp.float32),
        pltpu.VMEM((8, 8, 128), jnp.float32),
        pltpu.SemaphoreType.DMA,
        pltpu.SemaphoreType.DMA,
        pltpu.SemaphoreType.DMA,
        pltpu.SemaphoreType.DMA,
    ],
    compiler_params=pltpu.CompilerParams(
        use_tc_tiling_on_sc=False, needs_layout_passes=False
    ),
)
def _emb_lookup(
    xt_hbm, lut_hbm, out_hbm, idx_v, rows0, rows1, tb0, tb1, g0, g1, t0, t1
):
    w = lax.axis_index("s") * 2 + lax.axis_index("c")

    pltpu.sync_copy(xt_hbm.at[:, pl.ds(w * BLK, BLK)], idx_v)

    def start_gather(s, buf, gsem):
        pltpu.make_async_copy(lut_hbm.at[idx_v.at[s]], buf, gsem).start()

    def wait_gather(buf, gsem):
        pltpu.make_async_copy(lut_hbm.at[idx_v.at[0]], buf, gsem).wait()

    def start_write(s, tbuf, tsem):
        pltpu.make_async_copy(tbuf, out_hbm.at[s, :, w], tsem).start()

    def wait_write(tbuf, tsem):
        pltpu.make_async_copy(tbuf, out_hbm.at[0, :, w], tsem).wait()

    iota = lax.iota(jnp.int32, LANES)
    row_sel = [iota + lg * LANES for lg in range(8)]

    def transpose_scale(rows, tbuf):
        @plsc.parallel_loop(0, D_MODEL, unroll=8)
        def _(d):
            dt = d // 8
            ds = d % 8
            col = jnp.full((LANES,), d, jnp.int32)
            for lg in range(8):
                v = plsc.load_gather(rows, [row_sel[lg], col])
                tbuf[dt, ds, pl.ds(lg * LANES, LANES)] = v * SCALE

    start_gather(0, rows0, g0)
    start_gather(1, rows1, g1)

    wait_gather(rows0, g0)
    transpose_scale(rows0, tb0)
    start_gather(2, rows0, g0)
    start_write(0, tb0, t0)

    wait_gather(rows1, g1)
    transpose_scale(rows1, tb1)
    start_gather(3, rows1, g1)
    start_write(1, tb1, t1)

    def pair_body(k, _):
        s0 = 2 * k
        wait_gather(rows0, g0)
        wait_write(tb0, t0)
        transpose_scale(rows0, tb0)
        start_gather(s0 + 2, rows0, g0)
        start_write(s0, tb0, t0)

        wait_gather(rows1, g1)
        wait_write(tb1, t1)
        transpose_scale(rows1, tb1)
        start_gather(s0 + 3, rows1, g1)
        start_write(s0 + 1, tb1, t1)
        return 0

    lax.fori_loop(1, SEQ // 2 - 1, pair_body, 0)

    wait_gather(rows0, g0)
    wait_write(tb0, t0)
    transpose_scale(rows0, tb0)
    start_write(SEQ - 2, tb0, t0)
    wait_gather(rows1, g1)
    wait_write(tb1, t1)
    transpose_scale(rows1, tb1)
    start_write(SEQ - 1, tb1, t1)
    wait_write(tb0, t0)
    wait_write(tb1, t1)


def kernel(x, lut):
    xt = jnp.transpose(x).astype(jnp.int32)
    out_phys = _emb_lookup(xt, lut)
    return jnp.transpose(out_phys, (2, 4, 0, 1, 3)).reshape(BATCH, SEQ, D_MODEL)


# speedup vs baseline: 2.5816x; 1.6416x over previous
import functools
import math

import jax
import jax.numpy as jnp
from jax import lax
from jax.experimental import pallas as pl
from jax.experimental.pallas import tpu as pltpu
from jax.experimental.pallas import tpu_sc as plsc

D_MODEL = 64
SCALE = math.sqrt(D_MODEL)
NUM_WORKERS = 32
SEQ = 200
BATCH = 4096
BLK = 128
LANES = 16


@functools.partial(
    pl.kernel,
    mesh=plsc.VectorSubcoreMesh(core_axis_name="c", subcore_axis_name="s"),
    out_type=jax.ShapeDtypeStruct((SEQ, 8, NUM_WORKERS, 8, 128), jnp.float32),
    scratch_types=[
        pltpu.VMEM((SEQ, BLK), jnp.int32),
        pltpu.VMEM((BLK, D_MODEL), jnp.float32),
        pltpu.VMEM((BLK, D_MODEL), jnp.float32),
        pltpu.VMEM((D_MODEL, 137), jnp.float32),
        pltpu.VMEM((D_MODEL, 137), jnp.float32),
        pltpu.SemaphoreType.DMA,
        pltpu.SemaphoreType.DMA,
        pltpu.SemaphoreType.DMA,
        pltpu.SemaphoreType.DMA,
    ],
    compiler_params=pltpu.CompilerParams(
        use_tc_tiling_on_sc=False, needs_layout_passes=False
    ),
)
def _emb_lookup(
    xt_hbm, lut_hbm, out_hbm, idx_v, rows0, rows1, tb0, tb1, g0, g1, t0, t1
):
    w = lax.axis_index("s") * 2 + lax.axis_index("c")

    pltpu.sync_copy(xt_hbm.at[:, pl.ds(w * BLK, BLK)], idx_v)

    def start_gather(s, buf, gsem):
        pltpu.make_async_copy(lut_hbm.at[idx_v.at[s]], buf, gsem).start()

    def wait_gather(buf, gsem):
        pltpu.make_async_copy(lut_hbm.at[idx_v.at[0]], buf, gsem).wait()

    def start_write(s, tbuf, tsem):
        for dt in range(8):
            pltpu.make_async_copy(
                tbuf.at[pl.ds(dt * 8, 8), pl.ds(0, 128)],
                out_hbm.at[s, dt, w],
                tsem,
            ).start()

    def wait_write(tbuf, tsem):
        for dt in range(8):
            pltpu.make_async_copy(
                tbuf.at[pl.ds(0, 8), pl.ds(0, 128)],
                out_hbm.at[0, dt, w],
                tsem,
            ).wait()

    iota = lax.iota(jnp.int32, LANES)
    d_sel = [dg * LANES + iota for dg in range(D_MODEL // LANES)]

    def transpose_scale(rows, tbuf):
        @plsc.parallel_loop(0, BLK, unroll=4)
        def _(b):
            bvec = jnp.full((LANES,), b, jnp.int32)
            for dg in range(D_MODEL // LANES):
                v = rows[b, pl.ds(dg * LANES, LANES)]
                plsc.store_scatter(tbuf, [d_sel[dg], bvec], v * SCALE)

    start_gather(0, rows0, g0)
    start_gather(1, rows1, g1)

    wait_gather(rows0, g0)
    transpose_scale(rows0, tb0)
    start_gather(2, rows0, g0)
    start_write(0, tb0, t0)

    wait_gather(rows1, g1)
    transpose_scale(rows1, tb1)
    start_gather(3, rows1, g1)
    start_write(1, tb1, t1)

    def pair_body(k, _):
        s0 = 2 * k
        wait_gather(rows0, g0)
        wait_write(tb0, t0)
        transpose_scale(rows0, tb0)
        start_gather(s0 + 2, rows0, g0)
        start_write(s0, tb0, t0)

        wait_gather(rows1, g1)
        wait_write(tb1, t1)
        transpose_scale(rows1, tb1)
        start_gather(s0 + 3, rows1, g1)
        start_write(s0 + 1, tb1, t1)
        return 0

    lax.fori_loop(1, SEQ // 2 - 1, pair_body, 0)

    wait_gather(rows0, g0)
    wait_write(tb0, t0)
    transpose_scale(rows0, tb0)
    start_write(SEQ - 2, tb0, t0)
    wait_gather(rows1, g1)
    wait_write(tb1, t1)
    transpose_scale(rows1, tb1)
    start_write(SEQ - 1, tb1, t1)
    wait_write(tb0, t0)
    wait_write(tb1, t1)


def kernel(x, lut):
    xt = jnp.transpose(x).astype(jnp.int32)
    out_phys = _emb_lookup(xt, lut)
    return jnp.transpose(out_phys, (2, 4, 0, 1, 3)).reshape(BATCH, SEQ, D_MODEL)


# speedup vs baseline: 3.8367x; 1.4861x over previous
import functools
import math

import jax
import jax.numpy as jnp
from jax import lax
from jax.experimental import pallas as pl
from jax.experimental.pallas import tpu as pltpu
from jax.experimental.pallas import tpu_sc as plsc
from jax.experimental import layout as jex_layout

D_MODEL = 64
SCALE = math.sqrt(D_MODEL)
NUM_WORKERS = 32
SEQ = 200
BATCH = 4096
BLK = 128
LANES = 16


@functools.partial(
    pl.kernel,
    mesh=plsc.VectorSubcoreMesh(core_axis_name="c", subcore_axis_name="s"),
    out_type=jax.ShapeDtypeStruct((SEQ, 8, NUM_WORKERS, 8, 128), jnp.float32),
    scratch_types=[
        pltpu.VMEM((SEQ, BLK), jnp.int32),
        pltpu.VMEM((BLK, D_MODEL), jnp.float32),
        pltpu.VMEM((BLK, D_MODEL), jnp.float32),
        pltpu.VMEM((D_MODEL, 137), jnp.float32),
        pltpu.VMEM((D_MODEL, 137), jnp.float32),
        pltpu.SemaphoreType.DMA,
        pltpu.SemaphoreType.DMA,
        pltpu.SemaphoreType.DMA,
        pltpu.SemaphoreType.DMA,
    ],
    compiler_params=pltpu.CompilerParams(
        use_tc_tiling_on_sc=False, needs_layout_passes=False
    ),
)
def _emb_lookup(
    xt_hbm, lut_hbm, out_hbm, idx_v, rows0, rows1, tb0, tb1, g0, g1, t0, t1
):
    w = lax.axis_index("s") * 2 + lax.axis_index("c")

    pltpu.sync_copy(xt_hbm.at[:, pl.ds(w * BLK, BLK)], idx_v)

    def start_gather(s, buf, gsem):
        pltpu.make_async_copy(lut_hbm.at[idx_v.at[s]], buf, gsem).start()

    def wait_gather(buf, gsem):
        pltpu.make_async_copy(lut_hbm.at[idx_v.at[0]], buf, gsem).wait()

    def start_write(s, tbuf, tsem):
        for dt in range(8):
            pltpu.make_async_copy(
                tbuf.at[pl.ds(dt * 8, 8), pl.ds(0, 128)],
                out_hbm.at[s, dt, w],
                tsem,
            ).start()

    def wait_write(tbuf, tsem):
        for dt in range(8):
            pltpu.make_async_copy(
                tbuf.at[pl.ds(0, 8), pl.ds(0, 128)],
                out_hbm.at[0, dt, w],
                tsem,
            ).wait()

    iota = lax.iota(jnp.int32, LANES)
    d_sel = [dg * LANES + iota for dg in range(D_MODEL // LANES)]

    def transpose_scale(rows, tbuf):
        @plsc.parallel_loop(0, BLK, unroll=4)
        def _(b):
            bvec = jnp.full((LANES,), b, jnp.int32)
            for dg in range(D_MODEL // LANES):
                v = rows[b, pl.ds(dg * LANES, LANES)]
                plsc.store_scatter(tbuf, [d_sel[dg], bvec], v * SCALE)

    start_gather(0, rows0, g0)
    start_gather(1, rows1, g1)

    wait_gather(rows0, g0)
    transpose_scale(rows0, tb0)
    start_gather(2, rows0, g0)
    start_write(0, tb0, t0)

    wait_gather(rows1, g1)
    transpose_scale(rows1, tb1)
    start_gather(3, rows1, g1)
    start_write(1, tb1, t1)

    def pair_body(k, _):
        s0 = 2 * k
        wait_gather(rows0, g0)
        wait_write(tb0, t0)
        transpose_scale(rows0, tb0)
        start_gather(s0 + 2, rows0, g0)
        start_write(s0, tb0, t0)

        wait_gather(rows1, g1)
        wait_write(tb1, t1)
        transpose_scale(rows1, tb1)
        start_gather(s0 + 3, rows1, g1)
        start_write(s0 + 1, tb1, t1)
        return 0

    lax.fori_loop(1, SEQ // 2 - 1, pair_body, 0)

    wait_gather(rows0, g0)
    wait_write(tb0, t0)
    transpose_scale(rows0, tb0)
    start_write(SEQ - 2, tb0, t0)
    wait_gather(rows1, g1)
    wait_write(tb1, t1)
    transpose_scale(rows1, tb1)
    start_write(SEQ - 1, tb1, t1)
    wait_write(tb0, t0)
    wait_write(tb1, t1)


def kernel(x, lut):
    xt = jnp.transpose(x).astype(jnp.int32)
    lut_lin = jex_layout.with_layout_constraint(
        lut, jex_layout.Layout(major_to_minor=(0, 1), tiling=((8,),))
    )
    out_phys = _emb_lookup(xt, lut_lin)
    return jnp.transpose(out_phys, (2, 4, 0, 1, 3)).reshape(BATCH, SEQ, D_MODEL)
